# baseline (speedup 1.0000x reference)
import functools

import jax
import jax.numpy as jnp
from jax import lax
from jax.experimental import pallas as pl
from jax.experimental.pallas import tpu as pltpu
from jax.experimental.pallas import tpu_sc as plsc

S = 100000
A = 8
R = 2
L = 16
B = 16384

NC = 2
NS = 16
NW = NC * NS
C = B // NW
NG = C // L
NSUB = C // 128


def _sc_body(st_h, ac_h, rw_h, gt_h, p2_h, e2_h, v4_h, w4_h, out_h,
             sbuf, abuf, rbuf, pidx, eidx, vwidx, vvidx, kbuf, cbuf,
             pg, eg, vg, wg, gt, outv, sem_a, sem_b):
    wid = lax.axis_index("s") * NC + lax.axis_index("c")
    wbase = pl.multiple_of(wid * C, C)

    gt_cp = pltpu.async_copy(gt_h.at[:, pl.ds(wbase, C)], gt, sem_b)
    pltpu.sync_copy(st_h.at[pl.ds(wbase, C)], sbuf)
    pltpu.sync_copy(ac_h.at[pl.ds(wbase, C)], abuf)
    pltpu.sync_copy(rw_h.at[pl.ds(wbase, C)], rbuf)

    def idx_body(i, carry):
        o = pl.multiple_of(i * L, L)
        s = jnp.maximum(sbuf[pl.ds(o, L)], 0)
        a = jnp.maximum(abuf[pl.ds(o, L)], 0)
        r = jnp.maximum(rbuf[pl.ds(o, L)], 0)
        pidx[pl.ds(o, L)] = s * (A * R) + a * R + r
        eidx[pl.ds(o, L)] = s * A + a
        sbuf[pl.ds(o, L)] = s
        return carry

    lax.fori_loop(0, NG, idx_body, 0)

    p_cps, e_cps = [], []
    for j in range(NSUB):
        sl = pl.ds(j * 128, 128)
        p_cps.append(pltpu.async_copy(p2_h.at[pidx.at[sl]], pg.at[sl], sem_a))
        e_cps.append(pltpu.async_copy(e2_h.at[eidx.at[sl]], eg.at[sl], sem_a))
    gt_cp.wait()
    for cp in p_cps:
        cp.wait()

    def pa_body(g, carry):
        o = pl.multiple_of(g * L, L)
        rows = lax.iota(jnp.int32, L) + o
        zt = [plsc.load_gather(pg, [rows, jnp.full((L,), l, jnp.int32)])
              for l in range(L)]
        zmax = zt[0]
        for l in range(1, L):
            zmax = jnp.maximum(zmax, zt[l])
        se = None
        best = None
        kvec = None
        for l in range(L):
            e = jnp.exp(zt[l] - zmax)
            se = e if se is None else se + e
            y = zt[l] + gt[l, pl.ds(o, L)]
            if l == 0:
                best = y
                kvec = jnp.zeros((L,), jnp.int32)
            else:
                upd = y > best
                best = jnp.where(upd, y, best)
                kvec = jnp.where(upd, jnp.full((L,), l, jnp.int32), kvec)
        zk = plsc.load_gather(pg, [rows, kvec])
        p = jnp.exp(zk - zmax) * (1.0 / se)
        cbuf[pl.ds(o, L)] = (1.0 + p) - p
        kbuf[pl.ds(o, L)] = kvec
        svec = sbuf[pl.ds(o, L)]
        vwidx[pl.ds(o, L)] = svec * L + kvec
        vvidx[pl.ds(o, L)] = svec * 4 + jax.lax.shift_right_logical(kvec, 2)
        return carry

    lax.fori_loop(0, NG, pa_body, 0)

    v_cps, w_cps = [], []
    for j in range(NSUB):
        sl = pl.ds(j * 128, 128)
        v_cps.append(pltpu.async_copy(v4_h.at[vvidx.at[sl]], vg.at[sl], sem_b))
        w_cps.append(pltpu.async_copy(w4_h.at[vwidx.at[sl]], wg.at[sl], sem_b))
    for cp in e_cps:
        cp.wait()
    for cp in v_cps + w_cps:
        cp.wait()

    def pb_body(g, carry):
        o = pl.multiple_of(g * L, L)
        rows = lax.iota(jnp.int32, L) + o
        cvec = cbuf[pl.ds(o, L)]
        kvec = kbuf[pl.ds(o, L)]
        vcol = (kvec & 3) * 2
        outv[0, pl.ds(o, L)] = cvec * plsc.load_gather(vg, [rows, vcol])
        outv[1, pl.ds(o, L)] = cvec * plsc.load_gather(vg, [rows, vcol + 1])
        for j in range(A):
            outv[2 + j, pl.ds(o, L)] = cvec * plsc.load_gather(
                wg, [rows, jnp.full((L,), j, jnp.int32)])
        outv[10, pl.ds(o, L)] = cvec * plsc.load_gather(eg, [rows, kvec])
        outv[11, pl.ds(o, L)] = cvec * plsc.load_gather(eg, [rows, kvec + L])
        return carry

    lax.fori_loop(0, NG, pb_body, 0)

    pltpu.sync_copy(outv, out_h.at[:, pl.ds(wbase, C)])


_sc_call = pl.kernel(
    _sc_body,
    out_type=jax.ShapeDtypeStruct((12, B), jnp.float32),
    mesh=plsc.VectorSubcoreMesh(core_axis_name="c", subcore_axis_name="s"),
    compiler_params=pltpu.CompilerParams(needs_layout_passes=False,
                                         use_tc_tiling_on_sc=False),
    scratch_types=[
        pltpu.VMEM((C,), jnp.int32),
        pltpu.VMEM((C,), jnp.int32),
        pltpu.VMEM((C,), jnp.int32),
        pltpu.VMEM((C,), jnp.int32),
        pltpu.VMEM((C,), jnp.int32),
        pltpu.VMEM((C,), jnp.int32),
        pltpu.VMEM((C,), jnp.int32),
        pltpu.VMEM((C,), jnp.int32),
        pltpu.VMEM((C,), jnp.float32),
        pltpu.VMEM((C, L), jnp.float32),
        pltpu.VMEM((C, R * L), jnp.float32),
        pltpu.VMEM((C, 8), jnp.float32),
        pltpu.VMEM((C, A), jnp.float32),
        pltpu.VMEM((L, C), jnp.float32),
        pltpu.VMEM((12, C), jnp.float32),
        pltpu.SemaphoreType.DMA,
        pltpu.SemaphoreType.DMA,
    ],
)


def _tc_body(comp_ref, act_ref, rew_ref, out_ref):
    t = comp_ref[...]
    a = jnp.maximum(act_ref[...], 0)
    r = jnp.maximum(rew_ref[...], 0)

    vl0, vl1 = t[0:1], t[1:2]
    vmax = jnp.maximum(vl0, vl1)
    vlse = vmax + jnp.log(jnp.exp(vl0 - vmax) + jnp.exp(vl1 - vmax))
    vloss = vlse - jnp.where(r == 0, vl0, vl1)

    al = t[2:10]
    amax = jnp.max(al, axis=0, keepdims=True)
    alse = amax + jnp.log(jnp.sum(jnp.exp(al - amax), axis=0, keepdims=True))
    apick = al[0:1]
    for j in range(1, A):
        apick = jnp.where(a == j, al[j:j + 1], apick)
    aloss = alse - apick

    ns0, ns1 = t[10:11], t[11:12]
    m = jnp.maximum(ns0, ns1)
    nlse = m + jnp.log(0.5 * jnp.exp(ns0 - m) + 0.5 * jnp.exp(ns1 - m))
    eloss = nlse - jnp.where(r == 0, ns0, ns1)

    inv_b = jnp.float32(1.0 / B)
    sv = jnp.sum(vloss, axis=1, keepdims=True)
    sa = jnp.sum(aloss, axis=1, keepdims=True)
    se = jnp.sum(eloss, axis=1, keepdims=True)
    out_ref[...] = sv * inv_b + sa * inv_b + se * inv_b


_tc_call = pl.pallas_call(
    _tc_body,
    out_shape=jax.ShapeDtypeStruct((1, 1), jnp.float32),
)


def kernel(states, actions, rewards, mask, posterior_net, value_net, policy_w,
           energy_net):
    del mask
    gum_t = jax.random.gumbel(jax.random.key(42), (B, L), jnp.float32).T
    p2 = posterior_net.reshape(S * A * R, L)
    e2 = energy_net.reshape(S * A, R * L)
    v4 = value_net.reshape(S * 4, 8)
    w4 = policy_w.reshape(S * L, A)
    comp = _sc_call(states, actions, rewards, gum_t, p2, e2, v4, w4)
    out = _tc_call(comp, actions.reshape(1, B), rewards.reshape(1, B))
    return out[0, 0]


# speedup vs baseline: 11.8460x; 11.8460x over previous
import jax
import jax.numpy as jnp
from jax import lax
from jax.experimental import pallas as pl
from jax.experimental.pallas import tpu as pltpu
from jax.experimental.pallas import tpu_sc as plsc

S = 100000
A = 8
R = 2
L = 16
B = 16384

NC = 2
NS = 16
NW = NC * NS
C = B // NW
NG = C // L
SUB = 128
NSUB = C // SUB
GS = SUB // L
SROW = S // L


def _sc_body(st_h, ac_h, rw_h, gt_h, pf_h, ef_h, vf_h, wf_h, out_h,
             sbuf, abuf, rbuf, srow_b, col_b, kbuf, cbuf,
             gt, pidx, pg, vidx, vg, widx, wg, eidx, eg, outv,
             sem_g, sem_p, sem_v, sem_w, sem_e):
    wid = lax.axis_index("s") * NC + lax.axis_index("c")
    wbase = pl.multiple_of(wid * C, C)

    gt_cp = pltpu.async_copy(gt_h.at[:, pl.ds(wbase, C)], gt, sem_g)
    pltpu.sync_copy(st_h.at[pl.ds(wbase, C)], sbuf)
    pltpu.sync_copy(ac_h.at[pl.ds(wbase, C)], abuf)
    pltpu.sync_copy(rw_h.at[pl.ds(wbase, C)], rbuf)

    def prep_body(i, carry):
        o = pl.multiple_of(i * L, L)
        s = jnp.maximum(sbuf[pl.ds(o, L)], 0)
        abuf[pl.ds(o, L)] = jnp.maximum(abuf[pl.ds(o, L)], 0)
        rbuf[pl.ds(o, L)] = jnp.maximum(rbuf[pl.ds(o, L)], 0)
        srow_b[pl.ds(o, L)] = lax.shift_right_logical(s, 4)
        col_b[pl.ds(o, L)] = s & (L - 1)
        return carry

    lax.fori_loop(0, NG, prep_body, 0)
    gt_cp.wait()

    for sub in range(NSUB):
        ob = sub * SUB

        def pidx_body(g, carry):
            o = pl.multiple_of(ob + g * L, L)
            oj = pl.multiple_of(g * L, L)
            base = (abuf[pl.ds(o, L)] * R + rbuf[pl.ds(o, L)]) * L
            srow = srow_b[pl.ds(o, L)]
            for l in range(L):
                pidx[pl.ds(l * SUB + oj, L)] = (base + l) * SROW + srow
            return carry

        lax.fori_loop(0, GS, pidx_body, 0)

        cps = [pltpu.async_copy(pf_h.at[pidx.at[pl.ds(l * SUB, SUB)]],
                                pg.at[pl.ds(l * SUB, SUB)], sem_p)
               for l in range(L)]
        for cp in cps:
            cp.wait()

        def pa_body(g, carry):
            o = pl.multiple_of(ob + g * L, L)
            oj = pl.multiple_of(g * L, L)
            jvec = lax.iota(jnp.int32, L) + oj
            colv = col_b[pl.ds(o, L)]
            zt = [plsc.load_gather(pg, [jvec + l * SUB, colv])
                  for l in range(L)]
            zmax = zt[0]
            for l in range(1, L):
                zmax = jnp.maximum(zmax, zt[l])
            se = None
            best = None
            kvec = None
            for l in range(L):
                e = jnp.exp(zt[l] - zmax)
                se = e if se is None else se + e
                y = zt[l] + gt[l, pl.ds(o, L)]
                if l == 0:
                    best = y
                    kvec = jnp.zeros((L,), jnp.int32)
                else:
                    upd = y > best
                    best = jnp.where(upd, y, best)
                    kvec = jnp.where(upd, jnp.full((L,), l, jnp.int32), kvec)
            zk = plsc.load_gather(pg, [kvec * SUB + jvec, colv])
            p = jnp.exp(zk - zmax) * (1.0 / se)
            cbuf[pl.ds(o, L)] = (1.0 + p) - p
            kbuf[pl.ds(o, L)] = kvec
            return carry

        lax.fori_loop(0, GS, pa_body, 0)

    for sub in range(NSUB):
        ob = sub * SUB

        def p2idx_body(g, carry):
            o = pl.multiple_of(ob + g * L, L)
            oj = pl.multiple_of(g * L, L)
            kvec = kbuf[pl.ds(o, L)]
            srow = srow_b[pl.ds(o, L)]
            avec = abuf[pl.ds(o, L)]
            for r in range(R):
                vidx[pl.ds(r * SUB + oj, L)] = (kvec * R + r) * SROW + srow
                eidx[pl.ds(r * SUB + oj, L)] = \
                    ((avec * R + r) * L + kvec) * SROW + srow
            for a in range(A):
                widx[pl.ds(a * SUB + oj, L)] = (kvec * A + a) * SROW + srow
            return carry

        lax.fori_loop(0, GS, p2idx_body, 0)

        cps = [pltpu.async_copy(vf_h.at[vidx.at[pl.ds(r * SUB, SUB)]],
                                vg.at[pl.ds(r * SUB, SUB)], sem_v)
               for r in range(R)]
        cps += [pltpu.async_copy(ef_h.at[eidx.at[pl.ds(r * SUB, SUB)]],
                                 eg.at[pl.ds(r * SUB, SUB)], sem_e)
                for r in range(R)]
        cps += [pltpu.async_copy(wf_h.at[widx.at[pl.ds(a * SUB, SUB)]],
                                 wg.at[pl.ds(a * SUB, SUB)], sem_w)
                for a in range(A)]
        for cp in cps:
            cp.wait()

        def pb_body(g, carry):
            o = pl.multiple_of(ob + g * L, L)
            oj = pl.multiple_of(g * L, L)
            jvec = lax.iota(jnp.int32, L) + oj
            colv = col_b[pl.ds(o, L)]
            cvec = cbuf[pl.ds(o, L)]
            for r in range(R):
                outv[r, pl.ds(o, L)] = cvec * plsc.load_gather(
                    vg, [jvec + r * SUB, colv])
            for a in range(A):
                outv[2 + a, pl.ds(o, L)] = cvec * plsc.load_gather(
                    wg, [jvec + a * SUB, colv])
            for r in range(R):
                outv[10 + r, pl.ds(o, L)] = cvec * plsc.load_gather(
                    eg, [jvec + r * SUB, colv])
            return carry

        lax.fori_loop(0, GS, pb_body, 0)

    pltpu.sync_copy(outv, out_h.at[:, pl.ds(wbase, C)])


_sc_call = pl.kernel(
    _sc_body,
    out_type=jax.ShapeDtypeStruct((12, B), jnp.float32),
    mesh=plsc.VectorSubcoreMesh(core_axis_name="c", subcore_axis_name="s"),
    compiler_params=pltpu.CompilerParams(needs_layout_passes=False,
                                         use_tc_tiling_on_sc=False),
    scratch_types=[
        pltpu.VMEM((C,), jnp.int32),
        pltpu.VMEM((C,), jnp.int32),
        pltpu.VMEM((C,), jnp.int32),
        pltpu.VMEM((C,), jnp.int32),
        pltpu.VMEM((C,), jnp.int32),
        pltpu.VMEM((C,), jnp.int32),
        pltpu.VMEM((C,), jnp.float32),
        pltpu.VMEM((L, C), jnp.float32),
        pltpu.VMEM((L * SUB,), jnp.int32),
        pltpu.VMEM((L * SUB, L), jnp.float32),
        pltpu.VMEM((R * SUB,), jnp.int32),
        pltpu.VMEM((R * SUB, L), jnp.float32),
        pltpu.VMEM((A * SUB,), jnp.int32),
        pltpu.VMEM((A * SUB, L), jnp.float32),
        pltpu.VMEM((R * SUB,), jnp.int32),
        pltpu.VMEM((R * SUB, L), jnp.float32),
        pltpu.VMEM((12, C), jnp.float32),
        pltpu.SemaphoreType.DMA,
        pltpu.SemaphoreType.DMA,
        pltpu.SemaphoreType.DMA,
        pltpu.SemaphoreType.DMA,
        pltpu.SemaphoreType.DMA,
    ],
)


def _tc_body(comp_ref, act_ref, rew_ref, out_ref):
    t = comp_ref[...]
    a = jnp.maximum(act_ref[...], 0)
    r = jnp.maximum(rew_ref[...], 0)

    vl0, vl1 = t[0:1], t[1:2]
    vmax = jnp.maximum(vl0, vl1)
    vlse = vmax + jnp.log(jnp.exp(vl0 - vmax) + jnp.exp(vl1 - vmax))
    vloss = vlse - jnp.where(r == 0, vl0, vl1)

    al = t[2:10]
    amax = jnp.max(al, axis=0, keepdims=True)
    alse = amax + jnp.log(jnp.sum(jnp.exp(al - amax), axis=0, keepdims=True))
    apick = al[0:1]
    for j in range(1, A):
        apick = jnp.where(a == j, al[j:j + 1], apick)
    aloss = alse - apick

    ns0, ns1 = t[10:11], t[11:12]
    m = jnp.maximum(ns0, ns1)
    nlse = m + jnp.log(0.5 * jnp.exp(ns0 - m) + 0.5 * jnp.exp(ns1 - m))
    eloss = nlse - jnp.where(r == 0, ns0, ns1)

    inv_b = jnp.float32(1.0 / B)
    sv = jnp.sum(vloss, axis=1, keepdims=True)
    sa = jnp.sum(aloss, axis=1, keepdims=True)
    sen = jnp.sum(eloss, axis=1, keepdims=True)
    out_ref[...] = sv * inv_b + sa * inv_b + sen * inv_b


_tc_call = pl.pallas_call(
    _tc_body,
    out_shape=jax.ShapeDtypeStruct((1, 1), jnp.float32),
)


def kernel(states, actions, rewards, mask, posterior_net, value_net, policy_w,
           energy_net):
    del mask
    gum_t = jax.random.gumbel(jax.random.key(42), (B, L), jnp.float32).T
    pf = jnp.transpose(posterior_net, (1, 2, 3, 0)).reshape(A * R * L * SROW, L)
    ef = jnp.transpose(energy_net, (1, 2, 3, 0)).reshape(A * R * L * SROW, L)
    vf = jnp.transpose(value_net, (1, 2, 0)).reshape(L * R * SROW, L)
    wf = jnp.transpose(policy_w, (1, 2, 0)).reshape(L * A * SROW, L)
    comp = _sc_call(states, actions, rewards, gum_t, pf, ef, vf, wf)
    out = _tc_call(comp, actions.reshape(1, B), rewards.reshape(1, B))
    return out[0, 0]


# speedup vs baseline: 12.3346x; 1.0412x over previous
import jax
import jax.numpy as jnp
from jax import lax
from jax.experimental import pallas as pl
from jax.experimental.pallas import tpu as pltpu
from jax.experimental.pallas import tpu_sc as plsc

S = 100000
A = 8
R = 2
L = 16
B = 16384

NC = 2
NS = 16
NW = NC * NS
C = B // NW
NG = C // L
SUB = 128
NSUB = C // SUB
GS = SUB // L
SROW = S // L


def _sc_a_body(st_h, ac_h, rw_h, gt_h, pf_h, k_out, c_out,
               sbuf, abuf, rbuf, srow_b, col_b, kbuf, cbuf,
               gt, pidx, pg, sem_g, sem_p):
    wid = lax.axis_index("s") * NC + lax.axis_index("c")
    wbase = pl.multiple_of(wid * C, C)

    gt_cp = pltpu.async_copy(gt_h.at[:, pl.ds(wbase, C)], gt, sem_g)
    pltpu.sync_copy(st_h.at[pl.ds(wbase, C)], sbuf)
    pltpu.sync_copy(ac_h.at[pl.ds(wbase, C)], abuf)
    pltpu.sync_copy(rw_h.at[pl.ds(wbase, C)], rbuf)

    def prep_body(i, carry):
        o = pl.multiple_of(i * L, L)
        s = jnp.maximum(sbuf[pl.ds(o, L)], 0)
        abuf[pl.ds(o, L)] = jnp.maximum(abuf[pl.ds(o, L)], 0)
        rbuf[pl.ds(o, L)] = jnp.maximum(rbuf[pl.ds(o, L)], 0)
        srow_b[pl.ds(o, L)] = lax.shift_right_logical(s, 4)
        col_b[pl.ds(o, L)] = s & (L - 1)
        return carry

    lax.fori_loop(0, NG, prep_body, 0)
    gt_cp.wait()

    for sub in range(NSUB):
        ob = sub * SUB

        def pidx_body(g, carry):
            o = pl.multiple_of(ob + g * L, L)
            oj = pl.multiple_of(g * L, L)
            base = (abuf[pl.ds(o, L)] * R + rbuf[pl.ds(o, L)]) * L
            srow = srow_b[pl.ds(o, L)]
            for l in range(L):
                pidx[pl.ds(l * SUB + oj, L)] = (base + l) * SROW + srow
            return carry

        lax.fori_loop(0, GS, pidx_body, 0)

        cps = [pltpu.async_copy(pf_h.at[pidx.at[pl.ds(l * SUB, SUB)]],
                                pg.at[pl.ds(l * SUB, SUB)], sem_p)
               for l in range(L)]
        for cp in cps:
            cp.wait()

        def pa_body(g, carry):
            o = pl.multiple_of(ob + g * L, L)
            oj = pl.multiple_of(g * L, L)
            jvec = lax.iota(jnp.int32, L) + oj
            colv = col_b[pl.ds(o, L)]
            zt = [plsc.load_gather(pg, [jvec + l * SUB, colv])
                  for l in range(L)]
            zmax = zt[0]
            for l in range(1, L):
                zmax = jnp.maximum(zmax, zt[l])
            se = None
            best = None
            kvec = None
            for l in range(L):
                e = jnp.exp(zt[l] - zmax)
                se = e if se is None else se + e
                y = zt[l] + gt[l, pl.ds(o, L)]
                if l == 0:
                    best = y
                    kvec = jnp.zeros((L,), jnp.int32)
                else:
                    upd = y > best
                    best = jnp.where(upd, y, best)
                    kvec = jnp.where(upd, jnp.full((L,), l, jnp.int32), kvec)
            zk = plsc.load_gather(pg, [kvec * SUB + jvec, colv])
            p = jnp.exp(zk - zmax) * (1.0 / se)
            cbuf[pl.ds(o, L)] = (1.0 + p) - p
            kbuf[pl.ds(o, L)] = kvec
            return carry

        lax.fori_loop(0, GS, pa_body, 0)

    pltpu.sync_copy(kbuf, k_out.at[pl.ds(wbase, C)])
    pltpu.sync_copy(cbuf, c_out.at[pl.ds(wbase, C)])


def _sc_b_body(st_h, ac_h, k_h, c_h, ef_h, vf_h, wf_h, out_h,
               sbuf, abuf, srow_b, col_b, kbuf, cbuf,
               vidx, vg, widx, wg, eidx, eg, outv,
               sem_v, sem_w, sem_e):
    wid = lax.axis_index("s") * NC + lax.axis_index("c")
    wbase = pl.multiple_of(wid * C, C)

    pltpu.sync_copy(st_h.at[pl.ds(wbase, C)], sbuf)
    pltpu.sync_copy(ac_h.at[pl.ds(wbase, C)], abuf)
    pltpu.sync_copy(k_h.at[pl.ds(wbase, C)], kbuf)
    pltpu.sync_copy(c_h.at[pl.ds(wbase, C)], cbuf)

    def prep_body(i, carry):
        o = pl.multiple_of(i * L, L)
        s = jnp.maximum(sbuf[pl.ds(o, L)], 0)
        abuf[pl.ds(o, L)] = jnp.maximum(abuf[pl.ds(o, L)], 0)
        srow_b[pl.ds(o, L)] = lax.shift_right_logical(s, 4)
        col_b[pl.ds(o, L)] = s & (L - 1)
        return carry

    lax.fori_loop(0, NG, prep_body, 0)

    for sub in range(NSUB):
        ob = sub * SUB

        def p2idx_body(g, carry):
            o = pl.multiple_of(ob + g * L, L)
            oj = pl.multiple_of(g * L, L)
            kvec = kbuf[pl.ds(o, L)]
            srow = srow_b[pl.ds(o, L)]
            avec = abuf[pl.ds(o, L)]
            for r in range(R):
                vidx[pl.ds(r * SUB + oj, L)] = (kvec * R + r) * SROW + srow
                eidx[pl.ds(r * SUB + oj, L)] = \
                    ((avec * R + r) * L + kvec) * SROW + srow
            for a in range(A):
                widx[pl.ds(a * SUB + oj, L)] = (kvec * A + a) * SROW + srow
            return carry

        lax.fori_loop(0, GS, p2idx_body, 0)

        cps = [pltpu.async_copy(vf_h.at[vidx.at[pl.ds(r * SUB, SUB)]],
                                vg.at[pl.ds(r * SUB, SUB)], sem_v)
               for r in range(R)]
        cps += [pltpu.async_copy(ef_h.at[eidx.at[pl.ds(r * SUB, SUB)]],
                                 eg.at[pl.ds(r * SUB, SUB)], sem_e)
                for r in range(R)]
        cps += [pltpu.async_copy(wf_h.at[widx.at[pl.ds(a * SUB, SUB)]],
                                 wg.at[pl.ds(a * SUB, SUB)], sem_w)
                for a in range(A)]
        for cp in cps:
            cp.wait()

        def pb_body(g, carry):
            o = pl.multiple_of(ob + g * L, L)
            oj = pl.multiple_of(g * L, L)
            jvec = lax.iota(jnp.int32, L) + oj
            colv = col_b[pl.ds(o, L)]
            cvec = cbuf[pl.ds(o, L)]
            for r in range(R):
                outv[r, pl.ds(o, L)] = cvec * plsc.load_gather(
                    vg, [jvec + r * SUB, colv])
            for a in range(A):
                outv[2 + a, pl.ds(o, L)] = cvec * plsc.load_gather(
                    wg, [jvec + a * SUB, colv])
            for r in range(R):
                outv[10 + r, pl.ds(o, L)] = cvec * plsc.load_gather(
                    eg, [jvec + r * SUB, colv])
            return carry

        lax.fori_loop(0, GS, pb_body, 0)

    pltpu.sync_copy(outv, out_h.at[:, pl.ds(wbase, C)])


_sc_a_call = pl.kernel(
    _sc_a_body,
    out_type=[jax.ShapeDtypeStruct((B,), jnp.int32),
              jax.ShapeDtypeStruct((B,), jnp.float32)],
    mesh=plsc.VectorSubcoreMesh(core_axis_name="c", subcore_axis_name="s"),
    compiler_params=pltpu.CompilerParams(needs_layout_passes=False,
                                         use_tc_tiling_on_sc=False),
    scratch_types=[
        pltpu.VMEM((C,), jnp.int32),
        pltpu.VMEM((C,), jnp.int32),
        pltpu.VMEM((C,), jnp.int32),
        pltpu.VMEM((C,), jnp.int32),
        pltpu.VMEM((C,), jnp.int32),
        pltpu.VMEM((C,), jnp.int32),
        pltpu.VMEM((C,), jnp.float32),
        pltpu.VMEM((L, C), jnp.float32),
        pltpu.VMEM((L * SUB,), jnp.int32),
        pltpu.VMEM((L * SUB, L), jnp.float32),
        pltpu.SemaphoreType.DMA,
        pltpu.SemaphoreType.DMA,
    ],
)

_sc_b_call = pl.kernel(
    _sc_b_body,
    out_type=jax.ShapeDtypeStruct((12, B), jnp.float32),
    mesh=plsc.VectorSubcoreMesh(core_axis_name="c", subcore_axis_name="s"),
    compiler_params=pltpu.CompilerParams(needs_layout_passes=False,
                                         use_tc_tiling_on_sc=False),
    scratch_types=[
        pltpu.VMEM((C,), jnp.int32),
        pltpu.VMEM((C,), jnp.int32),
        pltpu.VMEM((C,), jnp.int32),
        pltpu.VMEM((C,), jnp.int32),
        pltpu.VMEM((C,), jnp.int32),
        pltpu.VMEM((C,), jnp.float32),
        pltpu.VMEM((R * SUB,), jnp.int32),
        pltpu.VMEM((R * SUB, L), jnp.float32),
        pltpu.VMEM((A * SUB,), jnp.int32),
        pltpu.VMEM((A * SUB, L), jnp.float32),
        pltpu.VMEM((R * SUB,), jnp.int32),
        pltpu.VMEM((R * SUB, L), jnp.float32),
        pltpu.VMEM((12, C), jnp.float32),
        pltpu.SemaphoreType.DMA,
        pltpu.SemaphoreType.DMA,
        pltpu.SemaphoreType.DMA,
    ],
)


def _tc_body(comp_ref, act_ref, rew_ref, out_ref):
    t = comp_ref[...]
    a = jnp.maximum(act_ref[...], 0)
    r = jnp.maximum(rew_ref[...], 0)

    vl0, vl1 = t[0:1], t[1:2]
    vmax = jnp.maximum(vl0, vl1)
    vlse = vmax + jnp.log(jnp.exp(vl0 - vmax) + jnp.exp(vl1 - vmax))
    vloss = vlse - jnp.where(r == 0, vl0, vl1)

    al = t[2:10]
    amax = jnp.max(al, axis=0, keepdims=True)
    alse = amax + jnp.log(jnp.sum(jnp.exp(al - amax), axis=0, keepdims=True))
    apick = al[0:1]
    for j in range(1, A):
        apick = jnp.where(a == j, al[j:j + 1], apick)
    aloss = alse - apick

    ns0, ns1 = t[10:11], t[11:12]
    m = jnp.maximum(ns0, ns1)
    nlse = m + jnp.log(0.5 * jnp.exp(ns0 - m) + 0.5 * jnp.exp(ns1 - m))
    eloss = nlse - jnp.where(r == 0, ns0, ns1)

    inv_b = jnp.float32(1.0 / B)
    sv = jnp.sum(vloss, axis=1, keepdims=True)
    sa = jnp.sum(aloss, axis=1, keepdims=True)
    sen = jnp.sum(eloss, axis=1, keepdims=True)
    out_ref[...] = sv * inv_b + sa * inv_b + sen * inv_b


_tc_call = pl.pallas_call(
    _tc_body,
    out_shape=jax.ShapeDtypeStruct((1, 1), jnp.float32),
)


def kernel(states, actions, rewards, mask, posterior_net, value_net, policy_w,
           energy_net):
    del mask
    gum_t = jax.random.gumbel(jax.random.key(42), (B, L), jnp.float32).T
    pf = jnp.transpose(posterior_net, (1, 2, 3, 0)).reshape(A * R * L * SROW, L)
    ef = jnp.transpose(energy_net, (1, 2, 3, 0)).reshape(A * R * L * SROW, L)
    vf = jnp.transpose(value_net, (1, 2, 0)).reshape(L * R * SROW, L)
    wf = jnp.transpose(policy_w, (1, 2, 0)).reshape(L * A * SROW, L)
    kv, cv = _sc_a_call(states, actions, rewards, gum_t, pf)
    comp = _sc_b_call(states, actions, kv, cv, ef, vf, wf)
    out = _tc_call(comp, actions.reshape(1, B), rewards.reshape(1, B))
    return out[0, 0]
